# baseline (speedup 1.0000x reference)
import functools

import jax
import jax.numpy as jnp
from jax.experimental import pallas as pl
from jax.experimental.pallas import tpu as pltpu

D = 1024
E = 8
K = 2
DFF = 4 * D


def _route_body(x_ref, rw_ref, nw_ref, nb_ref, xn_ref, gate_ref):
    x = x_ref[...]
    mu = jnp.mean(x, axis=-1, keepdims=True)
    var = jnp.mean(jnp.square(x), axis=-1, keepdims=True) - jnp.square(mu)
    xn = (x - mu) * jax.lax.rsqrt(var + 1e-5) * nw_ref[...] + nb_ref[...]
    xn_ref[...] = xn
    logits = jax.lax.dot_general(
        xn, rw_ref[...], (((1,), (1,)), ((), ())),
        preferred_element_type=jnp.float32,
        precision=jax.lax.Precision.HIGHEST)
    probs = jax.nn.softmax(logits / 1.5, axis=-1)
    iota = jax.lax.broadcasted_iota(jnp.int32, probs.shape, 1)
    i0 = jnp.argmax(probs, axis=-1)[:, None]
    w0 = jnp.max(probs, axis=-1, keepdims=True)
    masked = jnp.where(iota == i0, -jnp.inf, probs)
    i1 = jnp.argmax(masked, axis=-1)[:, None]
    w1 = jnp.max(masked, axis=-1, keepdims=True)
    gate_ref[...] = jnp.where(iota == i0, w0, 0.0) + jnp.where(iota == i1, w1, 0.0)


def _ffn_body(xn_ref, gate_ref, x_ref, w1_ref, w2_ref, out_ref):
    e = pl.program_id(1)
    f = pl.program_id(2)
    h = jax.lax.dot_general(
        xn_ref[...], w1_ref[0], (((1,), (1,)), ((), ())),
        preferred_element_type=jnp.float32)
    h = h * jax.nn.sigmoid(h)
    y = jax.lax.dot_general(
        h, w2_ref[0], (((1,), (1,)), ((), ())),
        preferred_element_type=jnp.float32)
    gate = gate_ref[...]
    eiota = jax.lax.broadcasted_iota(jnp.int32, gate.shape, 1)
    g = jnp.sum(jnp.where(eiota == e, gate, 0.0), axis=1, keepdims=True)
    first = jnp.logical_and(e == 0, f == 0)

    @pl.when(first)
    def _():
        out_ref[...] = x_ref[...] + g * y

    @pl.when(jnp.logical_not(first))
    def _():
        out_ref[...] = out_ref[...] + g * y


@jax.jit
def kernel(x, router_W, W1, W2, norm_w, norm_b):
    B, T, _ = x.shape
    N = B * T
    xf = x.reshape(N, D)

    bm = 512
    xn, gate = pl.pallas_call(
        _route_body,
        grid=(N // bm,),
        in_specs=[
            pl.BlockSpec((bm, D), lambda i: (i, 0)),
            pl.BlockSpec((E, D), lambda i: (0, 0)),
            pl.BlockSpec((D,), lambda i: (0,)),
            pl.BlockSpec((D,), lambda i: (0,)),
        ],
        out_specs=[
            pl.BlockSpec((bm, D), lambda i: (i, 0)),
            pl.BlockSpec((bm, E), lambda i: (i, 0)),
        ],
        out_shape=[
            jax.ShapeDtypeStruct((N, D), jnp.float32),
            jax.ShapeDtypeStruct((N, E), jnp.float32),
        ],
        compiler_params=pltpu.CompilerParams(
            dimension_semantics=("parallel",)),
    )(xf, router_W, norm_w, norm_b)

    bf = 512
    out = pl.pallas_call(
        _ffn_body,
        grid=(N // bm, E, DFF // bf),
        in_specs=[
            pl.BlockSpec((bm, D), lambda t, e, f: (t, 0)),
            pl.BlockSpec((bm, E), lambda t, e, f: (t, 0)),
            pl.BlockSpec((bm, D), lambda t, e, f: (t, 0)),
            pl.BlockSpec((1, bf, D), lambda t, e, f: (e, f, 0)),
            pl.BlockSpec((1, D, bf), lambda t, e, f: (e, 0, f)),
        ],
        out_specs=pl.BlockSpec((bm, D), lambda t, e, f: (t, 0)),
        out_shape=jax.ShapeDtypeStruct((N, D), jnp.float32),
        compiler_params=pltpu.CompilerParams(
            dimension_semantics=("parallel", "arbitrary", "arbitrary")),
    )(xn, gate, xf, W1, W2)

    return out.reshape(B, T, D)


# speedup vs baseline: 1.4003x; 1.4003x over previous
import functools

import jax
import jax.numpy as jnp
from jax import lax
from jax.experimental import pallas as pl
from jax.experimental.pallas import tpu as pltpu
from jax.experimental.pallas import tpu_sc as plsc

D = 1024
E = 8
K = 2
DFF = 4 * D
N = 4096
A = N * K
BM = 256
NPAD = A + E * BM
NB = NPAD // BM


def _route_body(x_ref, rw_ref, nw_ref, nb_ref, xn_ref, eid_ref, wts_ref):
    x = x_ref[...]
    mu = jnp.mean(x, axis=-1, keepdims=True)
    var = jnp.mean(jnp.square(x - mu), axis=-1, keepdims=True)
    xn = (x - mu) * lax.rsqrt(var + 1e-5) * nw_ref[...] + nb_ref[...]
    xn_ref[...] = xn
    logits = lax.dot_general(
        xn, rw_ref[...], (((1,), (1,)), ((), ())),
        preferred_element_type=jnp.float32)
    probs = jax.nn.softmax(logits / 1.5, axis=-1)
    iota = lax.broadcasted_iota(jnp.int32, probs.shape, 1)
    i0 = jnp.argmax(probs, axis=-1)[:, None]
    w0 = jnp.max(probs, axis=-1, keepdims=True)
    masked = jnp.where(iota == i0, -jnp.inf, probs)
    i1 = jnp.argmax(masked, axis=-1)[:, None]
    w1 = jnp.max(masked, axis=-1, keepdims=True)
    eid_ref[...] = jnp.concatenate([i0, i1], axis=1)
    wts_ref[...] = jnp.concatenate([w0, w1], axis=1)


def _route(xf, router_W, norm_w, norm_b):
    bm = 512
    return pl.pallas_call(
        _route_body,
        grid=(N // bm,),
        in_specs=[
            pl.BlockSpec((bm, D), lambda i: (i, 0)),
            pl.BlockSpec((E, D), lambda i: (0, 0)),
            pl.BlockSpec((D,), lambda i: (0,)),
            pl.BlockSpec((D,), lambda i: (0,)),
        ],
        out_specs=[
            pl.BlockSpec((bm, D), lambda i: (i, 0)),
            pl.BlockSpec((bm, K), lambda i: (i, 0)),
            pl.BlockSpec((bm, K), lambda i: (i, 0)),
        ],
        out_shape=[
            jax.ShapeDtypeStruct((N, D), jnp.float32),
            jax.ShapeDtypeStruct((N, K), jnp.int32),
            jax.ShapeDtypeStruct((N, K), jnp.float32),
        ],
        compiler_params=pltpu.CompilerParams(
            dimension_semantics=("parallel",)),
    )(xf, router_W, norm_w, norm_b)


_NW = 32
_GCH = 64


@functools.lru_cache(maxsize=None)
def _make_sc_gather():
    mesh = plsc.VectorSubcoreMesh(core_axis_name="c", subcore_axis_name="s")

    @functools.partial(
        pl.kernel, mesh=mesh,
        out_type=jax.ShapeDtypeStruct((NPAD, D), jnp.float32),
        scratch_types=[
            pltpu.VMEM((_GCH,), jnp.int32),
            pltpu.VMEM((_GCH, D), jnp.float32),
            pltpu.SemaphoreType.DMA,
        ],
    )
    def sc_gather(src_hbm, idx_hbm, out_hbm, idxc, rows, sem):
        wid = lax.axis_index("s") * 2 + lax.axis_index("c")
        base = wid * (NPAD // _NW)
        for c in range(NPAD // _NW // _GCH):
            off = base + c * _GCH
            pltpu.sync_copy(idx_hbm.at[pl.ds(off, _GCH)], idxc)
            pltpu.async_copy(src_hbm.at[idxc], rows, sem).wait()
            pltpu.sync_copy(rows, out_hbm.at[pl.ds(off, _GCH)])

    return sc_gather


@functools.lru_cache(maxsize=None)
def _make_sc_combine():
    mesh = plsc.VectorSubcoreMesh(core_axis_name="c", subcore_axis_name="s")

    @functools.partial(
        pl.kernel, mesh=mesh,
        out_type=[jax.ShapeDtypeStruct((N, D), jnp.float32),
                  jax.ShapeDtypeStruct((N, D), jnp.float32)],
        scratch_types=[
            pltpu.VMEM((_GCH,), jnp.int32),
            pltpu.VMEM((_GCH, D), jnp.float32),
            pltpu.SemaphoreType.DMA,
        ],
    )
    def sc_combine(ys_hbm, p0_hbm, p1_hbm, y0_hbm, y1_hbm, idxc, rows, sem):
        wid = lax.axis_index("s") * 2 + lax.axis_index("c")
        base = wid * (N // _NW)
        for c in range(N // _NW // _GCH):
            off = base + c * _GCH
            for p_hbm, o_hbm in ((p0_hbm, y0_hbm), (p1_hbm, y1_hbm)):
                pltpu.sync_copy(p_hbm.at[pl.ds(off, _GCH)], idxc)
                pltpu.async_copy(ys_hbm.at[idxc], rows, sem).wait()
                pltpu.sync_copy(rows, o_hbm.at[pl.ds(off, _GCH)])

    return sc_combine


def _sc_gather(src, idx):
    return _make_sc_gather()(src, idx)


def _sc_combine(ys, p0, p1):
    return _make_sc_combine()(ys, p0, p1)


_FC = 512


def _ffn_body(be_ref, xs_ref, wrow_ref, w1_ref, w2_ref, ys_ref):
    acc = jnp.zeros((BM, D), jnp.float32)
    xs = xs_ref[...].astype(jnp.bfloat16)
    for fc in range(DFF // _FC):
        w1c = w1_ref[0, fc * _FC:(fc + 1) * _FC, :]
        h = lax.dot_general(xs, w1c, (((1,), (1,)), ((), ())),
                            preferred_element_type=jnp.float32)
        h = (h * jax.nn.sigmoid(h)).astype(jnp.bfloat16)
        w2c = w2_ref[0, :, fc * _FC:(fc + 1) * _FC]
        acc = acc + lax.dot_general(h, w2c, (((1,), (1,)), ((), ())),
                                    preferred_element_type=jnp.float32)
    ys_ref[...] = acc * wrow_ref[...][:, None]


def _ffn(blk_e, xs, wrow, W1, W2):
    grid_spec = pltpu.PrefetchScalarGridSpec(
        num_scalar_prefetch=1,
        grid=(NB,),
        in_specs=[
            pl.BlockSpec((BM, D), lambda b, be: (b, 0)),
            pl.BlockSpec((BM,), lambda b, be: (b,)),
            pl.BlockSpec((1, DFF, D), lambda b, be: (be[b], 0, 0)),
            pl.BlockSpec((1, D, DFF), lambda b, be: (be[b], 0, 0)),
        ],
        out_specs=pl.BlockSpec((BM, D), lambda b, be: (b, 0)),
    )
    return pl.pallas_call(
        _ffn_body,
        grid_spec=grid_spec,
        out_shape=jax.ShapeDtypeStruct((NPAD, D), jnp.float32),
        compiler_params=pltpu.CompilerParams(
            dimension_semantics=("arbitrary",),
            vmem_limit_bytes=100 * 1024 * 1024),
    )(blk_e, xs, wrow, W1, W2)


def _add_body(x_ref, y0_ref, y1_ref, out_ref):
    out_ref[...] = x_ref[...] + y0_ref[...] + y1_ref[...]


def _add(xf, y0, y1):
    bm = 512
    return pl.pallas_call(
        _add_body,
        grid=(N // bm,),
        in_specs=[pl.BlockSpec((bm, D), lambda i: (i, 0))] * 3,
        out_specs=pl.BlockSpec((bm, D), lambda i: (i, 0)),
        out_shape=jax.ShapeDtypeStruct((N, D), jnp.float32),
        compiler_params=pltpu.CompilerParams(
            dimension_semantics=("parallel",)),
    )(xf, y0, y1)


@jax.jit
def kernel(x, router_W, W1, W2, norm_w, norm_b):
    B, T, _ = x.shape
    xf = x.reshape(N, D)

    xn, eids, wts = _route(xf, router_W, norm_w, norm_b)

    flat_e = eids.reshape(A)
    flat_w = wts.reshape(A)
    order = jnp.argsort(flat_e, stable=True).astype(jnp.int32)
    ej = flat_e[order]
    counts = jnp.bincount(flat_e, length=E)
    starts = jnp.concatenate([jnp.zeros((1,), counts.dtype),
                              jnp.cumsum(counts)[:-1]])
    cap = ((counts + BM - 1) // BM) * BM
    pad_start = jnp.concatenate([jnp.zeros((1,), cap.dtype),
                                 jnp.cumsum(cap)[:-1]])
    pos = (pad_start[ej] + jnp.arange(A, dtype=jnp.int32)
           - starts[ej]).astype(jnp.int32)
    src_row = jnp.zeros((NPAD,), jnp.int32).at[pos].set(
        (order // K).astype(jnp.int32))
    wrow = jnp.zeros((NPAD,), jnp.float32).at[pos].set(flat_w[order])
    padpos = jnp.zeros((A,), jnp.int32).at[order].set(pos)
    p0 = padpos.reshape(N, K)[:, 0]
    p1 = padpos.reshape(N, K)[:, 1]
    blk_e = jnp.minimum(
        jnp.searchsorted(pad_start + cap, jnp.arange(NB) * BM,
                         side="right"),
        E - 1).astype(jnp.int32)

    xs = _sc_gather(xn, src_row)
    ys = _ffn(blk_e, xs, wrow, W1.astype(jnp.bfloat16),
              W2.astype(jnp.bfloat16))
    y0, y1 = _sc_combine(ys, p0, p1)
    out = _add(xf, y0, y1)
    return out.reshape(B, T, D)


# speedup vs baseline: 1.7345x; 1.2387x over previous
import functools

import jax
import jax.numpy as jnp
from jax import lax
from jax.experimental import pallas as pl
from jax.experimental.pallas import tpu as pltpu
from jax.experimental.pallas import tpu_sc as plsc

D = 1024
E = 8
K = 2
DFF = 4 * D
N = 4096
A = N * K
BM = 256
NPAD = A + E * BM
NB = NPAD // BM


def _route_body(x_ref, rw_ref, nw_ref, nb_ref, xn_ref, eid_ref, wts_ref):
    x = x_ref[...]
    mu = jnp.mean(x, axis=-1, keepdims=True)
    var = jnp.mean(jnp.square(x - mu), axis=-1, keepdims=True)
    xn = (x - mu) * lax.rsqrt(var + 1e-5) * nw_ref[...] + nb_ref[...]
    xn_ref[...] = xn
    logits = lax.dot_general(
        xn, rw_ref[...], (((1,), (1,)), ((), ())),
        preferred_element_type=jnp.float32)
    probs = jax.nn.softmax(logits / 1.5, axis=-1)
    iota = lax.broadcasted_iota(jnp.int32, probs.shape, 1)
    i0 = jnp.argmax(probs, axis=-1)[:, None]
    w0 = jnp.max(probs, axis=-1, keepdims=True)
    masked = jnp.where(iota == i0, -jnp.inf, probs)
    i1 = jnp.argmax(masked, axis=-1)[:, None]
    w1 = jnp.max(masked, axis=-1, keepdims=True)
    eid_ref[...] = jnp.concatenate([i0, i1], axis=1)
    wts_ref[...] = jnp.concatenate([w0, w1], axis=1)


def _route(xf, router_W, norm_w, norm_b):
    bm = 512
    return pl.pallas_call(
        _route_body,
        grid=(N // bm,),
        in_specs=[
            pl.BlockSpec((bm, D), lambda i: (i, 0)),
            pl.BlockSpec((E, D), lambda i: (0, 0)),
            pl.BlockSpec((D,), lambda i: (0,)),
            pl.BlockSpec((D,), lambda i: (0,)),
        ],
        out_specs=[
            pl.BlockSpec((bm, D), lambda i: (i, 0)),
            pl.BlockSpec((bm, K), lambda i: (i, 0)),
            pl.BlockSpec((bm, K), lambda i: (i, 0)),
        ],
        out_shape=[
            jax.ShapeDtypeStruct((N, D), jnp.float32),
            jax.ShapeDtypeStruct((N, K), jnp.int32),
            jax.ShapeDtypeStruct((N, K), jnp.float32),
        ],
        compiler_params=pltpu.CompilerParams(
            dimension_semantics=("parallel",)),
    )(xf, router_W, norm_w, norm_b)


_NW = 32
_GCH = 64


_GROWS = 40


@functools.lru_cache(maxsize=None)
def _make_sc_gather():
    mesh = plsc.VectorSubcoreMesh(core_axis_name="c", subcore_axis_name="s")
    bpw = NPAD // _NW
    nch = bpw // _GROWS

    @functools.partial(
        pl.kernel, mesh=mesh,
        out_type=jax.ShapeDtypeStruct((NPAD, D), jnp.float32),
        scratch_types=[
            pltpu.VMEM((bpw,), jnp.int32),
            pltpu.VMEM((_GROWS, D), jnp.float32),
            pltpu.VMEM((_GROWS, D), jnp.float32),
            pltpu.SemaphoreType.DMA,
            pltpu.SemaphoreType.DMA,
        ],
    )
    def sc_gather(src_hbm, idx_hbm, out_hbm, idxv, rows0, rows1, sem0, sem1):
        wid = lax.axis_index("s") * 2 + lax.axis_index("c")
        base = wid * bpw
        pltpu.sync_copy(idx_hbm.at[pl.ds(base, bpw)], idxv)
        bufs = (rows0, rows1)
        sems = (sem0, sem1)
        cps = []
        for c in range(nch):
            cps.append(pltpu.async_copy(
                src_hbm.at[idxv.at[pl.ds(c * _GROWS, _GROWS)]],
                bufs[c % 2], sems[c % 2]))
            if c > 0:
                cps[c - 1].wait()
                pltpu.sync_copy(bufs[(c - 1) % 2],
                                out_hbm.at[pl.ds(base + (c - 1) * _GROWS,
                                                 _GROWS)])
        cps[nch - 1].wait()
        pltpu.sync_copy(bufs[(nch - 1) % 2],
                        out_hbm.at[pl.ds(base + (nch - 1) * _GROWS, _GROWS)])

    return sc_gather


@functools.lru_cache(maxsize=None)
def _make_sc_combine():
    mesh = plsc.VectorSubcoreMesh(core_axis_name="c", subcore_axis_name="s")

    @functools.partial(
        pl.kernel, mesh=mesh,
        out_type=[jax.ShapeDtypeStruct((N, D), jnp.float32),
                  jax.ShapeDtypeStruct((N, D), jnp.float32)],
        scratch_types=[
            pltpu.VMEM((_GCH,), jnp.int32),
            pltpu.VMEM((_GCH, D), jnp.float32),
            pltpu.SemaphoreType.DMA,
        ],
    )
    def sc_combine(ys_hbm, p0_hbm, p1_hbm, y0_hbm, y1_hbm, idxc, rows, sem):
        wid = lax.axis_index("s") * 2 + lax.axis_index("c")
        base = wid * (N // _NW)
        for c in range(N // _NW // _GCH):
            off = base + c * _GCH
            for p_hbm, o_hbm in ((p0_hbm, y0_hbm), (p1_hbm, y1_hbm)):
                pltpu.sync_copy(p_hbm.at[pl.ds(off, _GCH)], idxc)
                pltpu.async_copy(ys_hbm.at[idxc], rows, sem).wait()
                pltpu.sync_copy(rows, o_hbm.at[pl.ds(off, _GCH)])

    return sc_combine


def _sc_gather(src, idx):
    return _make_sc_gather()(src, idx)


def _sc_combine(ys, p0, p1):
    return _make_sc_combine()(ys, p0, p1)


_FC = 512


_DH = DFF // 2


def _ffn_body_first(be_ref, xs_ref, w1_ref, w2_ref, ys_ref):
    acc = jnp.zeros((BM, D), jnp.float32)
    xs = xs_ref[...]
    for fc in range(_DH // _FC):
        w1c = w1_ref[0, fc * _FC:(fc + 1) * _FC, :]
        h = lax.dot_general(xs, w1c, (((1,), (1,)), ((), ())),
                            preferred_element_type=jnp.float32)
        h = h * jax.nn.sigmoid(h)
        w2c = w2_ref[0, :, fc * _FC:(fc + 1) * _FC]
        acc = acc + lax.dot_general(h, w2c, (((1,), (1,)), ((), ())),
                                    preferred_element_type=jnp.float32)
    ys_ref[...] = acc


def _ffn_body_second(be_ref, xs_ref, ysin_ref, wrow_ref, w1_ref, w2_ref,
                     ys_ref):
    acc = jnp.zeros((BM, D), jnp.float32)
    xs = xs_ref[...]
    for fc in range(_DH // _FC):
        w1c = w1_ref[0, fc * _FC:(fc + 1) * _FC, :]
        h = lax.dot_general(xs, w1c, (((1,), (1,)), ((), ())),
                            preferred_element_type=jnp.float32)
        h = h * jax.nn.sigmoid(h)
        w2c = w2_ref[0, :, fc * _FC:(fc + 1) * _FC]
        acc = acc + lax.dot_general(h, w2c, (((1,), (1,)), ((), ())),
                                    preferred_element_type=jnp.float32)
    ys_ref[...] = (ysin_ref[...] + acc) * wrow_ref[...][:, None]


def _ffn(blk_e, xs, wrow, W1, W2):
    cp = pltpu.CompilerParams(
        dimension_semantics=("arbitrary",),
        vmem_limit_bytes=100 * 1024 * 1024)
    gs1 = pltpu.PrefetchScalarGridSpec(
        num_scalar_prefetch=1,
        grid=(NB,),
        in_specs=[
            pl.BlockSpec((BM, D), lambda b, be: (b, 0)),
            pl.BlockSpec((1, _DH, D), lambda b, be: (be[b], 0, 0)),
            pl.BlockSpec((1, D, _DH), lambda b, be: (be[b], 0, 0)),
        ],
        out_specs=pl.BlockSpec((BM, D), lambda b, be: (b, 0)),
    )
    ys0 = pl.pallas_call(
        _ffn_body_first,
        grid_spec=gs1,
        out_shape=jax.ShapeDtypeStruct((NPAD, D), jnp.float32),
        compiler_params=cp,
    )(blk_e, xs, W1, W2)
    gs2 = pltpu.PrefetchScalarGridSpec(
        num_scalar_prefetch=1,
        grid=(NB,),
        in_specs=[
            pl.BlockSpec((BM, D), lambda b, be: (b, 0)),
            pl.BlockSpec((BM, D), lambda b, be: (b, 0)),
            pl.BlockSpec((BM,), lambda b, be: (b,)),
            pl.BlockSpec((1, _DH, D), lambda b, be: (be[b], 1, 0)),
            pl.BlockSpec((1, D, _DH), lambda b, be: (be[b], 0, 1)),
        ],
        out_specs=pl.BlockSpec((BM, D), lambda b, be: (b, 0)),
    )
    return pl.pallas_call(
        _ffn_body_second,
        grid_spec=gs2,
        out_shape=jax.ShapeDtypeStruct((NPAD, D), jnp.float32),
        compiler_params=cp,
    )(blk_e, xs, ys0, wrow, W1, W2)


def _add_body(x_ref, y0_ref, y1_ref, out_ref):
    out_ref[...] = x_ref[...] + y0_ref[...] + y1_ref[...]


def _add(xf, y0, y1):
    bm = 512
    return pl.pallas_call(
        _add_body,
        grid=(N // bm,),
        in_specs=[pl.BlockSpec((bm, D), lambda i: (i, 0))] * 3,
        out_specs=pl.BlockSpec((bm, D), lambda i: (i, 0)),
        out_shape=jax.ShapeDtypeStruct((N, D), jnp.float32),
        compiler_params=pltpu.CompilerParams(
            dimension_semantics=("parallel",)),
    )(xf, y0, y1)


@jax.jit
def kernel(x, router_W, W1, W2, norm_w, norm_b):
    B, T, _ = x.shape
    xf = x.reshape(N, D)

    xn, eids, wts = _route(xf, router_W, norm_w, norm_b)

    flat_e = eids.reshape(A)
    flat_w = wts.reshape(A)
    ej, order, wj = lax.sort(
        (flat_e, jnp.arange(A, dtype=jnp.int32), flat_w),
        num_keys=1, is_stable=True)
    oh = (ej[:, None] == jnp.arange(E, dtype=jnp.int32)[None, :])
    ohf = oh.astype(jnp.float32)
    counts = jnp.sum(oh, axis=0)
    starts = jnp.concatenate([jnp.zeros((1,), counts.dtype),
                              jnp.cumsum(counts)[:-1]])
    cap = ((counts + BM - 1) // BM) * BM
    pad_start = jnp.concatenate([jnp.zeros((1,), cap.dtype),
                                 jnp.cumsum(cap)[:-1]])
    shift = (pad_start - starts).astype(jnp.float32)
    pos = (jnp.arange(A, dtype=jnp.int32)
           + (ohf @ shift).astype(jnp.int32))
    src_row = (jnp.arange(NPAD, dtype=jnp.int32) % N).at[pos].set(
        (order // K).astype(jnp.int32))
    wrow = jnp.zeros((NPAD,), jnp.float32).at[pos].set(wj)
    padpos = jnp.zeros((A,), jnp.int32).at[order].set(pos)
    p0 = padpos.reshape(N, K)[:, 0]
    p1 = padpos.reshape(N, K)[:, 1]
    blk_e = jnp.minimum(
        jnp.searchsorted(pad_start + cap, jnp.arange(NB) * BM,
                         side="right"),
        E - 1).astype(jnp.int32)

    xs = _sc_gather(xn, src_row)
    ys = _ffn(blk_e, xs, wrow, W1.astype(jnp.bfloat16),
              W2.astype(jnp.bfloat16))
    y0, y1 = _sc_combine(ys, p0, p1)
    out = _add(xf, y0, y1)
    return out.reshape(B, T, D)


# speedup vs baseline: 2.0098x; 1.1587x over previous
import functools

import jax
import jax.numpy as jnp
from jax import lax
from jax.experimental import pallas as pl
from jax.experimental.pallas import tpu as pltpu
from jax.experimental.pallas import tpu_sc as plsc

D = 1024
E = 8
K = 2
DFF = 4 * D
N = 4096
A = N * K
BM = 256
NPAD = A + E * BM
NB = NPAD // BM


def _route_body(x_ref, rw_ref, nw_ref, nb_ref, xn_ref, eid_ref, wts_ref):
    x = x_ref[...]
    mu = jnp.mean(x, axis=-1, keepdims=True)
    var = jnp.mean(jnp.square(x - mu), axis=-1, keepdims=True)
    xn = (x - mu) * lax.rsqrt(var + 1e-5) * nw_ref[...] + nb_ref[...]
    xn_ref[...] = xn
    logits = lax.dot_general(
        xn, rw_ref[...], (((1,), (1,)), ((), ())),
        preferred_element_type=jnp.float32)
    probs = jax.nn.softmax(logits / 1.5, axis=-1)
    iota = lax.broadcasted_iota(jnp.int32, probs.shape, 1)
    i0 = jnp.argmax(probs, axis=-1)[:, None]
    w0 = jnp.max(probs, axis=-1, keepdims=True)
    masked = jnp.where(iota == i0, -jnp.inf, probs)
    i1 = jnp.argmax(masked, axis=-1)[:, None]
    w1 = jnp.max(masked, axis=-1, keepdims=True)
    eid_ref[...] = jnp.concatenate([i0, i1], axis=1)
    wts_ref[...] = jnp.concatenate([w0, w1], axis=1)


def _route(xf, router_W, norm_w, norm_b):
    bm = 512
    return pl.pallas_call(
        _route_body,
        grid=(N // bm,),
        in_specs=[
            pl.BlockSpec((bm, D), lambda i: (i, 0)),
            pl.BlockSpec((E, D), lambda i: (0, 0)),
            pl.BlockSpec((D,), lambda i: (0,)),
            pl.BlockSpec((D,), lambda i: (0,)),
        ],
        out_specs=[
            pl.BlockSpec((bm, D), lambda i: (i, 0)),
            pl.BlockSpec((bm, K), lambda i: (i, 0)),
            pl.BlockSpec((bm, K), lambda i: (i, 0)),
        ],
        out_shape=[
            jax.ShapeDtypeStruct((N, D), jnp.float32),
            jax.ShapeDtypeStruct((N, K), jnp.int32),
            jax.ShapeDtypeStruct((N, K), jnp.float32),
        ],
        compiler_params=pltpu.CompilerParams(
            dimension_semantics=("parallel",)),
    )(xf, router_W, norm_w, norm_b)


_NW = 32
_GCH = 64


_GROWS = 40


@functools.lru_cache(maxsize=None)
def _make_sc_gather():
    mesh = plsc.VectorSubcoreMesh(core_axis_name="c", subcore_axis_name="s")
    bpw = NPAD // _NW
    nch = bpw // _GROWS

    @functools.partial(
        pl.kernel, mesh=mesh,
        out_type=jax.ShapeDtypeStruct((NPAD, D), jnp.float32),
        scratch_types=[
            pltpu.VMEM((bpw,), jnp.int32),
            pltpu.VMEM((_GROWS, D), jnp.float32),
            pltpu.VMEM((_GROWS, D), jnp.float32),
            pltpu.SemaphoreType.DMA,
            pltpu.SemaphoreType.DMA,
        ],
    )
    def sc_gather(src_hbm, idx_hbm, out_hbm, idxv, rows0, rows1, sem0, sem1):
        wid = lax.axis_index("s") * 2 + lax.axis_index("c")
        base = wid * bpw
        pltpu.sync_copy(idx_hbm.at[pl.ds(base, bpw)], idxv)
        bufs = (rows0, rows1)
        sems = (sem0, sem1)
        cps = []
        for c in range(nch):
            cps.append(pltpu.async_copy(
                src_hbm.at[idxv.at[pl.ds(c * _GROWS, _GROWS)]],
                bufs[c % 2], sems[c % 2]))
            if c > 0:
                cps[c - 1].wait()
                pltpu.sync_copy(bufs[(c - 1) % 2],
                                out_hbm.at[pl.ds(base + (c - 1) * _GROWS,
                                                 _GROWS)])
        cps[nch - 1].wait()
        pltpu.sync_copy(bufs[(nch - 1) % 2],
                        out_hbm.at[pl.ds(base + (nch - 1) * _GROWS, _GROWS)])

    return sc_gather


@functools.lru_cache(maxsize=None)
def _make_sc_combine():
    mesh = plsc.VectorSubcoreMesh(core_axis_name="c", subcore_axis_name="s")

    @functools.partial(
        pl.kernel, mesh=mesh,
        out_type=[jax.ShapeDtypeStruct((N, D), jnp.float32),
                  jax.ShapeDtypeStruct((N, D), jnp.float32)],
        scratch_types=[
            pltpu.VMEM((_GCH,), jnp.int32),
            pltpu.VMEM((_GCH, D), jnp.float32),
            pltpu.SemaphoreType.DMA,
        ],
    )
    def sc_combine(ys_hbm, p0_hbm, p1_hbm, y0_hbm, y1_hbm, idxc, rows, sem):
        wid = lax.axis_index("s") * 2 + lax.axis_index("c")
        base = wid * (N // _NW)
        for c in range(N // _NW // _GCH):
            off = base + c * _GCH
            for p_hbm, o_hbm in ((p0_hbm, y0_hbm), (p1_hbm, y1_hbm)):
                pltpu.sync_copy(p_hbm.at[pl.ds(off, _GCH)], idxc)
                pltpu.async_copy(ys_hbm.at[idxc], rows, sem).wait()
                pltpu.sync_copy(rows, o_hbm.at[pl.ds(off, _GCH)])

    return sc_combine


def _sc_gather(src, idx):
    return _make_sc_gather()(src, idx)


def _sc_combine(ys, p0, p1):
    return _make_sc_combine()(ys, p0, p1)


_FC = 512


_DH = DFF // 2


def _ffn_body_first(be_ref, xs_ref, w1_ref, w2_ref, ys_ref):
    acc = jnp.zeros((BM, D), jnp.float32)
    xs = xs_ref[...]
    for fc in range(_DH // _FC):
        w1c = w1_ref[0, fc * _FC:(fc + 1) * _FC, :]
        h = lax.dot_general(xs, w1c, (((1,), (1,)), ((), ())),
                            preferred_element_type=jnp.float32)
        h = h * jax.nn.sigmoid(h)
        w2c = w2_ref[0, :, fc * _FC:(fc + 1) * _FC]
        acc = acc + lax.dot_general(h, w2c, (((1,), (1,)), ((), ())),
                                    preferred_element_type=jnp.float32)
    ys_ref[...] = acc


def _ffn_body_second(be_ref, xs_ref, ysin_ref, wrow_ref, w1_ref, w2_ref,
                     ys_ref):
    acc = jnp.zeros((BM, D), jnp.float32)
    xs = xs_ref[...]
    for fc in range(_DH // _FC):
        w1c = w1_ref[0, fc * _FC:(fc + 1) * _FC, :]
        h = lax.dot_general(xs, w1c, (((1,), (1,)), ((), ())),
                            preferred_element_type=jnp.float32)
        h = h * jax.nn.sigmoid(h)
        w2c = w2_ref[0, :, fc * _FC:(fc + 1) * _FC]
        acc = acc + lax.dot_general(h, w2c, (((1,), (1,)), ((), ())),
                                    preferred_element_type=jnp.float32)
    ys_ref[...] = (ysin_ref[...] + acc) * wrow_ref[...][:, None]


def _ffn(blk_e, xs, wrow, W1, W2):
    cp = pltpu.CompilerParams(
        dimension_semantics=("arbitrary",),
        vmem_limit_bytes=100 * 1024 * 1024)
    gs1 = pltpu.PrefetchScalarGridSpec(
        num_scalar_prefetch=1,
        grid=(NB,),
        in_specs=[
            pl.BlockSpec((BM, D), lambda b, be: (b, 0)),
            pl.BlockSpec((1, _DH, D), lambda b, be: (be[b], 0, 0)),
            pl.BlockSpec((1, D, _DH), lambda b, be: (be[b], 0, 0)),
        ],
        out_specs=pl.BlockSpec((BM, D), lambda b, be: (b, 0)),
    )
    ys0 = pl.pallas_call(
        _ffn_body_first,
        grid_spec=gs1,
        out_shape=jax.ShapeDtypeStruct((NPAD, D), jnp.float32),
        compiler_params=cp,
    )(blk_e, xs, W1, W2)
    gs2 = pltpu.PrefetchScalarGridSpec(
        num_scalar_prefetch=1,
        grid=(NB,),
        in_specs=[
            pl.BlockSpec((BM, D), lambda b, be: (b, 0)),
            pl.BlockSpec((BM, D), lambda b, be: (b, 0)),
            pl.BlockSpec((BM,), lambda b, be: (b,)),
            pl.BlockSpec((1, _DH, D), lambda b, be: (be[b], 1, 0)),
            pl.BlockSpec((1, D, _DH), lambda b, be: (be[b], 0, 1)),
        ],
        out_specs=pl.BlockSpec((BM, D), lambda b, be: (b, 0)),
    )
    return pl.pallas_call(
        _ffn_body_second,
        grid_spec=gs2,
        out_shape=jax.ShapeDtypeStruct((NPAD, D), jnp.float32),
        compiler_params=cp,
    )(blk_e, xs, ys0, wrow, W1, W2)


def _add_body(x_ref, y0_ref, y1_ref, out_ref):
    out_ref[...] = x_ref[...] + y0_ref[...] + y1_ref[...]


def _add(xf, y0, y1):
    bm = 512
    return pl.pallas_call(
        _add_body,
        grid=(N // bm,),
        in_specs=[pl.BlockSpec((bm, D), lambda i: (i, 0))] * 3,
        out_specs=pl.BlockSpec((bm, D), lambda i: (i, 0)),
        out_shape=jax.ShapeDtypeStruct((N, D), jnp.float32),
        compiler_params=pltpu.CompilerParams(
            dimension_semantics=("parallel",)),
    )(xf, y0, y1)


@jax.jit
def kernel(x, router_W, W1, W2, norm_w, norm_b):
    B, T, _ = x.shape
    xf = x.reshape(N, D)

    xn, eids, wts = _route(xf, router_W, norm_w, norm_b)

    flat_e = eids.reshape(A)
    flat_w = wts.reshape(A)
    ej, order, wj = lax.sort(
        (flat_e, jnp.arange(A, dtype=jnp.int32), flat_w),
        num_keys=1, is_stable=True)
    oh = (ej[:, None] == jnp.arange(E, dtype=jnp.int32)[None, :])
    ohf = oh.astype(jnp.float32)
    counts = jnp.sum(oh, axis=0)
    starts = jnp.concatenate([jnp.zeros((1,), counts.dtype),
                              jnp.cumsum(counts)[:-1]])
    cap = ((counts + BM - 1) // BM) * BM
    pad_start = jnp.concatenate([jnp.zeros((1,), cap.dtype),
                                 jnp.cumsum(cap)[:-1]])
    shift = (pad_start - starts).astype(jnp.float32)
    pos = (jnp.arange(A, dtype=jnp.int32)
           + (ohf @ shift).astype(jnp.int32))
    src_row = (jnp.arange(NPAD, dtype=jnp.int32) % N).at[pos].set(
        (order // K).astype(jnp.int32))
    wrow = jnp.zeros((NPAD,), jnp.float32).at[pos].set(wj)
    padpos = jnp.zeros((A,), jnp.int32).at[order].set(pos)
    p0 = padpos.reshape(N, K)[:, 0]
    p1 = padpos.reshape(N, K)[:, 1]
    blk_e = jnp.minimum(
        jnp.searchsorted(pad_start + cap, jnp.arange(NB) * BM,
                         side="right"),
        E - 1).astype(jnp.int32)

    xs = _sc_gather(xn, src_row)
    ys = _ffn(blk_e, xs, wrow, W1, W2)
    y0, y1 = _sc_combine(ys, p0, p1)
    out = _add(xf, y0, y1)
    return out.reshape(B, T, D)


# speedup vs baseline: 2.2615x; 1.1252x over previous
import functools

import jax
import jax.numpy as jnp
from jax import lax
from jax.experimental import pallas as pl
from jax.experimental.pallas import tpu as pltpu
from jax.experimental.pallas import tpu_sc as plsc

D = 1024
E = 8
K = 2
DFF = 4 * D
N = 4096
A = N * K
BM = 256
NPAD = A + E * BM
NB = NPAD // BM


def _route_body(x_ref, rw_ref, nw_ref, nb_ref, xn_ref, eid_ref, wts_ref):
    x = x_ref[...]
    mu = jnp.mean(x, axis=-1, keepdims=True)
    var = jnp.mean(jnp.square(x - mu), axis=-1, keepdims=True)
    xn = (x - mu) * lax.rsqrt(var + 1e-5) * nw_ref[...] + nb_ref[...]
    xn_ref[...] = xn
    logits = lax.dot_general(
        xn, rw_ref[...], (((1,), (1,)), ((), ())),
        preferred_element_type=jnp.float32)
    probs = jax.nn.softmax(logits / 1.5, axis=-1)
    iota = lax.broadcasted_iota(jnp.int32, probs.shape, 1)
    i0 = jnp.argmax(probs, axis=-1)[:, None]
    w0 = jnp.max(probs, axis=-1, keepdims=True)
    masked = jnp.where(iota == i0, -jnp.inf, probs)
    i1 = jnp.argmax(masked, axis=-1)[:, None]
    w1 = jnp.max(masked, axis=-1, keepdims=True)
    eid_ref[...] = jnp.concatenate([i0, i1], axis=1)
    wts_ref[...] = jnp.concatenate([w0, w1], axis=1)


def _route(xf, router_W, norm_w, norm_b):
    bm = 512
    return pl.pallas_call(
        _route_body,
        grid=(N // bm,),
        in_specs=[
            pl.BlockSpec((bm, D), lambda i: (i, 0)),
            pl.BlockSpec((E, D), lambda i: (0, 0)),
            pl.BlockSpec((D,), lambda i: (0,)),
            pl.BlockSpec((D,), lambda i: (0,)),
        ],
        out_specs=[
            pl.BlockSpec((bm, D), lambda i: (i, 0)),
            pl.BlockSpec((bm, K), lambda i: (i, 0)),
            pl.BlockSpec((bm, K), lambda i: (i, 0)),
        ],
        out_shape=[
            jax.ShapeDtypeStruct((N, D), jnp.float32),
            jax.ShapeDtypeStruct((N, K), jnp.int32),
            jax.ShapeDtypeStruct((N, K), jnp.float32),
        ],
        compiler_params=pltpu.CompilerParams(
            dimension_semantics=("parallel",)),
    )(xf, router_W, norm_w, norm_b)


_NW = 32
_GCH = 64


_GROWS = 40


@functools.lru_cache(maxsize=None)
def _make_sc_gather():
    mesh = plsc.VectorSubcoreMesh(core_axis_name="c", subcore_axis_name="s")
    bpw = NPAD // _NW
    nch = bpw // _GROWS

    @functools.partial(
        pl.kernel, mesh=mesh,
        out_type=jax.ShapeDtypeStruct((NPAD, D), jnp.float32),
        scratch_types=[
            pltpu.VMEM((bpw,), jnp.int32),
            pltpu.VMEM((_GROWS, D), jnp.float32),
            pltpu.VMEM((_GROWS, D), jnp.float32),
            pltpu.SemaphoreType.DMA,
            pltpu.SemaphoreType.DMA,
        ],
    )
    def sc_gather(src_hbm, idx_hbm, out_hbm, idxv, rows0, rows1, sem0, sem1):
        wid = lax.axis_index("s") * 2 + lax.axis_index("c")
        base = wid * bpw
        pltpu.sync_copy(idx_hbm.at[pl.ds(base, bpw)], idxv)
        bufs = (rows0, rows1)
        sems = (sem0, sem1)
        cps = []
        for c in range(nch):
            cps.append(pltpu.async_copy(
                src_hbm.at[idxv.at[pl.ds(c * _GROWS, _GROWS)]],
                bufs[c % 2], sems[c % 2]))
            if c > 0:
                cps[c - 1].wait()
                pltpu.sync_copy(bufs[(c - 1) % 2],
                                out_hbm.at[pl.ds(base + (c - 1) * _GROWS,
                                                 _GROWS)])
        cps[nch - 1].wait()
        pltpu.sync_copy(bufs[(nch - 1) % 2],
                        out_hbm.at[pl.ds(base + (nch - 1) * _GROWS, _GROWS)])

    return sc_gather


@functools.lru_cache(maxsize=None)
def _make_sc_combine():
    mesh = plsc.VectorSubcoreMesh(core_axis_name="c", subcore_axis_name="s")

    @functools.partial(
        pl.kernel, mesh=mesh,
        out_type=[jax.ShapeDtypeStruct((N, D), jnp.float32),
                  jax.ShapeDtypeStruct((N, D), jnp.float32)],
        scratch_types=[
            pltpu.VMEM((_GCH,), jnp.int32),
            pltpu.VMEM((_GCH, D), jnp.float32),
            pltpu.SemaphoreType.DMA,
        ],
    )
    def sc_combine(ys_hbm, p0_hbm, p1_hbm, y0_hbm, y1_hbm, idxc, rows, sem):
        wid = lax.axis_index("s") * 2 + lax.axis_index("c")
        base = wid * (N // _NW)
        for c in range(N // _NW // _GCH):
            off = base + c * _GCH
            for p_hbm, o_hbm in ((p0_hbm, y0_hbm), (p1_hbm, y1_hbm)):
                pltpu.sync_copy(p_hbm.at[pl.ds(off, _GCH)], idxc)
                pltpu.async_copy(ys_hbm.at[idxc], rows, sem).wait()
                pltpu.sync_copy(rows, o_hbm.at[pl.ds(off, _GCH)])

    return sc_combine


def _sc_gather(src, idx):
    return _make_sc_gather()(src, idx)


def _sc_combine(ys, p0, p1):
    return _make_sc_combine()(ys, p0, p1)


_FC = 512


_DH = DFF // 2


def _ffn_body_first(be_ref, xs_ref, w1_ref, w2_ref, ys_ref):
    acc = jnp.zeros((BM, D), jnp.float32)
    xs = xs_ref[...]
    for fc in range(_DH // _FC):
        w1c = w1_ref[0, fc * _FC:(fc + 1) * _FC, :]
        h = lax.dot_general(xs, w1c, (((1,), (1,)), ((), ())),
                            preferred_element_type=jnp.float32)
        h = h * jax.nn.sigmoid(h)
        w2c = w2_ref[0, :, fc * _FC:(fc + 1) * _FC]
        acc = acc + lax.dot_general(h, w2c, (((1,), (1,)), ((), ())),
                                    preferred_element_type=jnp.float32)
    ys_ref[...] = acc


def _ffn_body_second(be_ref, xs_ref, ysin_ref, wrow_ref, w1_ref, w2_ref,
                     ys_ref):
    acc = jnp.zeros((BM, D), jnp.float32)
    xs = xs_ref[...]
    for fc in range(_DH // _FC):
        w1c = w1_ref[0, fc * _FC:(fc + 1) * _FC, :]
        h = lax.dot_general(xs, w1c, (((1,), (1,)), ((), ())),
                            preferred_element_type=jnp.float32)
        h = h * jax.nn.sigmoid(h)
        w2c = w2_ref[0, :, fc * _FC:(fc + 1) * _FC]
        acc = acc + lax.dot_general(h, w2c, (((1,), (1,)), ((), ())),
                                    preferred_element_type=jnp.float32)
    ys_ref[...] = (ysin_ref[...] + acc) * wrow_ref[...][:, None]


def _ffn(blk_e, xs, wrow, W1, W2):
    cp = pltpu.CompilerParams(
        dimension_semantics=("arbitrary",),
        vmem_limit_bytes=100 * 1024 * 1024)
    gs1 = pltpu.PrefetchScalarGridSpec(
        num_scalar_prefetch=1,
        grid=(NB,),
        in_specs=[
            pl.BlockSpec((BM, D), lambda b, be: (b, 0)),
            pl.BlockSpec((1, _DH, D), lambda b, be: (be[b], 0, 0)),
            pl.BlockSpec((1, D, _DH), lambda b, be: (be[b], 0, 0)),
        ],
        out_specs=pl.BlockSpec((BM, D), lambda b, be: (b, 0)),
    )
    ys0 = pl.pallas_call(
        _ffn_body_first,
        grid_spec=gs1,
        out_shape=jax.ShapeDtypeStruct((NPAD, D), jnp.float32),
        compiler_params=cp,
    )(blk_e, xs, W1, W2)
    gs2 = pltpu.PrefetchScalarGridSpec(
        num_scalar_prefetch=1,
        grid=(NB,),
        in_specs=[
            pl.BlockSpec((BM, D), lambda b, be: (b, 0)),
            pl.BlockSpec((BM, D), lambda b, be: (b, 0)),
            pl.BlockSpec((BM,), lambda b, be: (b,)),
            pl.BlockSpec((1, _DH, D), lambda b, be: (be[b], 1, 0)),
            pl.BlockSpec((1, D, _DH), lambda b, be: (be[b], 0, 1)),
        ],
        out_specs=pl.BlockSpec((BM, D), lambda b, be: (b, 0)),
    )
    return pl.pallas_call(
        _ffn_body_second,
        grid_spec=gs2,
        out_shape=jax.ShapeDtypeStruct((NPAD, D), jnp.float32),
        compiler_params=cp,
    )(blk_e, xs, ys0, wrow, W1, W2)


def _add_body(x_ref, y0_ref, y1_ref, out_ref):
    out_ref[...] = x_ref[...] + y0_ref[...] + y1_ref[...]


def _add(xf, y0, y1):
    bm = 512
    return pl.pallas_call(
        _add_body,
        grid=(N // bm,),
        in_specs=[pl.BlockSpec((bm, D), lambda i: (i, 0))] * 3,
        out_specs=pl.BlockSpec((bm, D), lambda i: (i, 0)),
        out_shape=jax.ShapeDtypeStruct((N, D), jnp.float32),
        compiler_params=pltpu.CompilerParams(
            dimension_semantics=("parallel",)),
    )(xf, y0, y1)


@jax.jit
def kernel(x, router_W, W1, W2, norm_w, norm_b):
    B, T, _ = x.shape
    xf = x.reshape(N, D)

    xn, eids, wts = _route(xf, router_W, norm_w, norm_b)

    flat_e = eids.reshape(A)
    flat_w = wts.reshape(A)
    ej, order, wj = lax.sort(
        (flat_e, jnp.arange(A, dtype=jnp.int32), flat_w),
        num_keys=1, is_stable=True)
    tokj = (order // K).astype(jnp.int32)
    oh = (ej[:, None] == jnp.arange(E, dtype=jnp.int32)[None, :])
    counts = jnp.sum(oh, axis=0).astype(jnp.int32)
    starts = jnp.concatenate([jnp.zeros((1,), jnp.int32),
                              jnp.cumsum(counts)[:-1]])
    cap = ((counts + BM - 1) // BM) * BM
    pad_start = jnp.concatenate([jnp.zeros((1,), jnp.int32),
                                 jnp.cumsum(cap)[:-1]])
    shift = (pad_start - starts).astype(jnp.float32)
    pos = (jnp.arange(A, dtype=jnp.int32)
           + (oh.astype(jnp.float32) @ shift).astype(jnp.int32))
    _, pos_s = lax.sort((order, pos), num_keys=1)
    p0 = pos_s.reshape(N, K)[:, 0]
    p1 = pos_s.reshape(N, K)[:, 1]
    pp = jnp.arange(NPAD, dtype=jnp.int32)
    pad_end = pad_start + cap
    eslot = jnp.minimum(
        jnp.sum((pp[:, None] >= pad_end[None, :]).astype(jnp.int32),
                axis=1), E - 1)
    ohp = (eslot[:, None] == jnp.arange(E, dtype=jnp.int32)[None, :]
           ).astype(jnp.float32)
    mat = jnp.stack([pad_start, counts, starts], axis=1).astype(jnp.float32)
    g = ohp @ mat
    r = pp - g[:, 0].astype(jnp.int32)
    real = r < g[:, 1].astype(jnp.int32)
    jc = jnp.clip(g[:, 2].astype(jnp.int32) + r, 0, A - 1)
    src_row = jnp.where(real, tokj[jc], pp % N)
    wrow = jnp.where(real, wj[jc], 0.0)
    bstart = jnp.arange(NB, dtype=jnp.int32) * BM
    blk_e = jnp.minimum(
        jnp.sum((bstart[:, None] >= pad_end[None, :]).astype(jnp.int32),
                axis=1), E - 1)

    xs = _sc_gather(xn, src_row)
    ys = _ffn(blk_e, xs, wrow, W1, W2)
    y0, y1 = _sc_combine(ys, p0, p1)
    out = _add(xf, y0, y1)
    return out.reshape(B, T, D)


# speedup vs baseline: 2.2727x; 1.0049x over previous
import functools

import jax
import jax.numpy as jnp
from jax import lax
from jax.experimental import pallas as pl
from jax.experimental.pallas import tpu as pltpu
from jax.experimental.pallas import tpu_sc as plsc

D = 1024
E = 8
K = 2
DFF = 4 * D
N = 4096
A = N * K
BM = 256
NPAD = A + E * BM
NB = NPAD // BM


def _route_body(x_ref, rw_ref, nw_ref, nb_ref, xn_ref, eid_ref, wts_ref):
    x = x_ref[...]
    mu = jnp.mean(x, axis=-1, keepdims=True)
    var = jnp.mean(jnp.square(x - mu), axis=-1, keepdims=True)
    xn = (x - mu) * lax.rsqrt(var + 1e-5) * nw_ref[...] + nb_ref[...]
    xn_ref[...] = xn
    logits = lax.dot_general(
        xn, rw_ref[...], (((1,), (1,)), ((), ())),
        preferred_element_type=jnp.float32)
    probs = jax.nn.softmax(logits / 1.5, axis=-1)
    iota = lax.broadcasted_iota(jnp.int32, probs.shape, 1)
    i0 = jnp.argmax(probs, axis=-1)[:, None]
    w0 = jnp.max(probs, axis=-1, keepdims=True)
    masked = jnp.where(iota == i0, -jnp.inf, probs)
    i1 = jnp.argmax(masked, axis=-1)[:, None]
    w1 = jnp.max(masked, axis=-1, keepdims=True)
    eid_ref[...] = jnp.concatenate([i0, i1], axis=1)
    wts_ref[...] = jnp.concatenate([w0, w1], axis=1)


def _route(xf, router_W, norm_w, norm_b):
    bm = 512
    return pl.pallas_call(
        _route_body,
        grid=(N // bm,),
        in_specs=[
            pl.BlockSpec((bm, D), lambda i: (i, 0)),
            pl.BlockSpec((E, D), lambda i: (0, 0)),
            pl.BlockSpec((D,), lambda i: (0,)),
            pl.BlockSpec((D,), lambda i: (0,)),
        ],
        out_specs=[
            pl.BlockSpec((bm, D), lambda i: (i, 0)),
            pl.BlockSpec((bm, K), lambda i: (i, 0)),
            pl.BlockSpec((bm, K), lambda i: (i, 0)),
        ],
        out_shape=[
            jax.ShapeDtypeStruct((N, D), jnp.float32),
            jax.ShapeDtypeStruct((N, K), jnp.int32),
            jax.ShapeDtypeStruct((N, K), jnp.float32),
        ],
        compiler_params=pltpu.CompilerParams(
            dimension_semantics=("parallel",)),
    )(xf, router_W, norm_w, norm_b)


_NW = 32
_GCH = 64


_GROWS = 40


@functools.lru_cache(maxsize=None)
def _make_sc_gather():
    mesh = plsc.VectorSubcoreMesh(core_axis_name="c", subcore_axis_name="s")
    bpw = NPAD // _NW
    nch = bpw // _GROWS

    @functools.partial(
        pl.kernel, mesh=mesh,
        out_type=jax.ShapeDtypeStruct((NPAD, D), jnp.float32),
        scratch_types=[
            pltpu.VMEM((bpw,), jnp.int32),
            pltpu.VMEM((_GROWS, D), jnp.float32),
            pltpu.VMEM((_GROWS, D), jnp.float32),
            pltpu.SemaphoreType.DMA,
            pltpu.SemaphoreType.DMA,
        ],
    )
    def sc_gather(src_hbm, idx_hbm, out_hbm, idxv, rows0, rows1, sem0, sem1):
        wid = lax.axis_index("s") * 2 + lax.axis_index("c")
        base = wid * bpw
        pltpu.sync_copy(idx_hbm.at[pl.ds(base, bpw)], idxv)
        bufs = (rows0, rows1)
        sems = (sem0, sem1)
        cps = []
        for c in range(nch):
            cps.append(pltpu.async_copy(
                src_hbm.at[idxv.at[pl.ds(c * _GROWS, _GROWS)]],
                bufs[c % 2], sems[c % 2]))
            if c > 0:
                cps[c - 1].wait()
                pltpu.sync_copy(bufs[(c - 1) % 2],
                                out_hbm.at[pl.ds(base + (c - 1) * _GROWS,
                                                 _GROWS)])
        cps[nch - 1].wait()
        pltpu.sync_copy(bufs[(nch - 1) % 2],
                        out_hbm.at[pl.ds(base + (nch - 1) * _GROWS, _GROWS)])

    return sc_gather


@functools.lru_cache(maxsize=None)
def _make_sc_combine():
    mesh = plsc.VectorSubcoreMesh(core_axis_name="c", subcore_axis_name="s")

    @functools.partial(
        pl.kernel, mesh=mesh,
        out_type=[jax.ShapeDtypeStruct((N, D), jnp.float32),
                  jax.ShapeDtypeStruct((N, D), jnp.float32)],
        scratch_types=[
            pltpu.VMEM((_GCH,), jnp.int32),
            pltpu.VMEM((_GCH, D), jnp.float32),
            pltpu.SemaphoreType.DMA,
        ],
    )
    def sc_combine(ys_hbm, p0_hbm, p1_hbm, y0_hbm, y1_hbm, idxc, rows, sem):
        wid = lax.axis_index("s") * 2 + lax.axis_index("c")
        base = wid * (N // _NW)
        for c in range(N // _NW // _GCH):
            off = base + c * _GCH
            for p_hbm, o_hbm in ((p0_hbm, y0_hbm), (p1_hbm, y1_hbm)):
                pltpu.sync_copy(p_hbm.at[pl.ds(off, _GCH)], idxc)
                pltpu.async_copy(ys_hbm.at[idxc], rows, sem).wait()
                pltpu.sync_copy(rows, o_hbm.at[pl.ds(off, _GCH)])

    return sc_combine


def _sc_gather(src, idx):
    return _make_sc_gather()(src, idx)


def _sc_combine(ys, p0, p1):
    return _make_sc_combine()(ys, p0, p1)


_FC = 512


_DH = DFF // 2


def _ffn_body_first(be_ref, xs_ref, w1_ref, w2_ref, ys_ref):
    acc = jnp.zeros((BM, D), jnp.float32)
    xs = xs_ref[...]
    for fc in range(_DH // _FC):
        w1c = w1_ref[0, fc * _FC:(fc + 1) * _FC, :]
        h = lax.dot_general(xs, w1c, (((1,), (1,)), ((), ())),
                            preferred_element_type=jnp.float32)
        h = h * jax.nn.sigmoid(h)
        w2c = w2_ref[0, :, fc * _FC:(fc + 1) * _FC]
        acc = acc + lax.dot_general(h, w2c, (((1,), (1,)), ((), ())),
                                    preferred_element_type=jnp.float32)
    ys_ref[...] = acc


def _ffn_body_second(be_ref, xs_ref, ysin_ref, wrow_ref, w1_ref, w2_ref,
                     ys_ref):
    acc = jnp.zeros((BM, D), jnp.float32)
    xs = xs_ref[...]
    for fc in range(_DH // _FC):
        w1c = w1_ref[0, fc * _FC:(fc + 1) * _FC, :]
        h = lax.dot_general(xs, w1c, (((1,), (1,)), ((), ())),
                            preferred_element_type=jnp.float32)
        h = h * jax.nn.sigmoid(h)
        w2c = w2_ref[0, :, fc * _FC:(fc + 1) * _FC]
        acc = acc + lax.dot_general(h, w2c, (((1,), (1,)), ((), ())),
                                    preferred_element_type=jnp.float32)
    ys_ref[...] = (ysin_ref[...] + acc) * wrow_ref[...][:, None]


def _ffn(blk_e, xs, wrow, W1, W2):
    cp = pltpu.CompilerParams(
        dimension_semantics=("arbitrary",),
        vmem_limit_bytes=100 * 1024 * 1024)
    gs1 = pltpu.PrefetchScalarGridSpec(
        num_scalar_prefetch=1,
        grid=(NB,),
        in_specs=[
            pl.BlockSpec((BM, D), lambda b, be: (b, 0)),
            pl.BlockSpec((1, _DH, D), lambda b, be: (be[b], 0, 0)),
            pl.BlockSpec((1, D, _DH), lambda b, be: (be[b], 0, 0)),
        ],
        out_specs=pl.BlockSpec((BM, D), lambda b, be: (b, 0)),
    )
    ys0 = pl.pallas_call(
        _ffn_body_first,
        grid_spec=gs1,
        out_shape=jax.ShapeDtypeStruct((NPAD, D), jnp.float32),
        compiler_params=cp,
    )(blk_e, xs, W1, W2)
    gs2 = pltpu.PrefetchScalarGridSpec(
        num_scalar_prefetch=1,
        grid=(NB,),
        in_specs=[
            pl.BlockSpec((BM, D), lambda b, be: (b, 0)),
            pl.BlockSpec((BM, D), lambda b, be: (b, 0)),
            pl.BlockSpec((BM,), lambda b, be: (b,)),
            pl.BlockSpec((1, _DH, D), lambda b, be: (be[b], 1, 0)),
            pl.BlockSpec((1, D, _DH), lambda b, be: (be[b], 0, 1)),
        ],
        out_specs=pl.BlockSpec((BM, D), lambda b, be: (b, 0)),
    )
    return pl.pallas_call(
        _ffn_body_second,
        grid_spec=gs2,
        out_shape=jax.ShapeDtypeStruct((NPAD, D), jnp.float32),
        compiler_params=cp,
    )(blk_e, xs, ys0, wrow, W1, W2)


def _add_body(x_ref, y0_ref, y1_ref, out_ref):
    out_ref[...] = x_ref[...] + y0_ref[...] + y1_ref[...]


def _add(xf, y0, y1):
    bm = 512
    return pl.pallas_call(
        _add_body,
        grid=(N // bm,),
        in_specs=[pl.BlockSpec((bm, D), lambda i: (i, 0))] * 3,
        out_specs=pl.BlockSpec((bm, D), lambda i: (i, 0)),
        out_shape=jax.ShapeDtypeStruct((N, D), jnp.float32),
        compiler_params=pltpu.CompilerParams(
            dimension_semantics=("parallel",)),
    )(xf, y0, y1)


@jax.jit
def kernel(x, router_W, W1, W2, norm_w, norm_b):
    B, T, _ = x.shape
    xf = x.reshape(N, D)

    xn, eids, wts = _route(xf, router_W, norm_w, norm_b)

    flat_e = eids.reshape(A)
    flat_w = wts.reshape(A)
    ej, order, wj = lax.sort(
        (flat_e, jnp.arange(A, dtype=jnp.int32), flat_w),
        num_keys=1, is_stable=True)
    tokj = (order // K).astype(jnp.int32)
    oh = (ej[:, None] == jnp.arange(E, dtype=jnp.int32)[None, :])
    counts = jnp.sum(oh, axis=0).astype(jnp.int32)
    starts = jnp.concatenate([jnp.zeros((1,), jnp.int32),
                              jnp.cumsum(counts)[:-1]])
    cap = ((counts + BM - 1) // BM) * BM
    pad_start = jnp.concatenate([jnp.zeros((1,), jnp.int32),
                                 jnp.cumsum(cap)[:-1]])
    shift = pad_start - starts
    pos = (jnp.arange(A, dtype=jnp.int32)
           + jnp.sum(jnp.where(oh, shift[None, :], 0), axis=1))
    _, pos_s = lax.sort((order, pos), num_keys=1)
    p0 = pos_s.reshape(N, K)[:, 0]
    p1 = pos_s.reshape(N, K)[:, 1]
    pp = jnp.arange(NPAD, dtype=jnp.int32)
    pad_end = pad_start + cap
    eslot = jnp.minimum(
        jnp.sum((pp[:, None] >= pad_end[None, :]).astype(jnp.int32),
                axis=1), E - 1)
    ohp = eslot[:, None] == jnp.arange(E, dtype=jnp.int32)[None, :]
    r = pp - jnp.sum(jnp.where(ohp, pad_start[None, :], 0), axis=1)
    real = r < jnp.sum(jnp.where(ohp, counts[None, :], 0), axis=1)
    jc = jnp.clip(
        jnp.sum(jnp.where(ohp, starts[None, :], 0), axis=1) + r, 0, A - 1)
    src_row = jnp.where(real, tokj[jc], pp % N)
    wrow = jnp.where(real, wj[jc], 0.0)
    bstart = jnp.arange(NB, dtype=jnp.int32) * BM
    blk_e = jnp.minimum(
        jnp.sum((bstart[:, None] >= pad_end[None, :]).astype(jnp.int32),
                axis=1), E - 1)

    xs = _sc_gather(xn, src_row)
    ys = _ffn(blk_e, xs, wrow, W1, W2)
    y0, y1 = _sc_combine(ys, p0, p1)
    out = _add(xf, y0, y1)
    return out.reshape(B, T, D)
